# Initial kernel scaffold; baseline (speedup 1.0000x reference)
import jax
import jax.numpy as jnp
from jax.experimental import pallas as pl


def kernel(x, edge_index, mask, W, v, b):
    raise NotImplementedError("write your pallas kernel here")


# speedup vs baseline: 26.0396x; 26.0396x over previous
import functools

import jax
import jax.numpy as jnp
from jax import lax
from jax.experimental import pallas as pl
from jax.experimental.pallas import tpu as pltpu
from jax.experimental.pallas import tpu_sc as plsc

N = 10000
D = 128
K = 8
KP = 16
E = 320000

NC = 2
NS = 16
NW = NC * NS
EPW = E // NW
CHUNK = 80
NCHUNK = EPW // CHUNK
NP = 10240
RPS = NP // NS

BN = 1000
NB = N // BN


def _stage_a_body(x_ref, m_ref, w_ref, v_ref, q_ref, ut_ref):
    j = pl.program_id(0)

    @pl.when(j == 0)
    def _():
        u = lax.dot_general(w_ref[...], v_ref[...],
                            (((2,), (1,)), ((0,), (0,))),
                            preferred_element_type=jnp.float32)
        ut_ref[...] = jnp.concatenate(
            [u, jnp.zeros((KP - K, D), jnp.float32)], axis=0)

    p = lax.dot_general(x_ref[...], ut_ref[...], (((1,), (1,)), ((), ())),
                        preferred_element_type=jnp.float32)
    q_ref[...] = p * m_ref[...]


_stage_a = pl.pallas_call(
    _stage_a_body,
    grid=(NB,),
    in_specs=[
        pl.BlockSpec((BN, D), lambda j: (j, 0)),
        pl.BlockSpec((BN, KP), lambda j: (j, 0)),
        pl.BlockSpec((K, D, D), lambda j: (0, 0, 0)),
        pl.BlockSpec((K, D), lambda j: (0, 0)),
    ],
    out_specs=pl.BlockSpec((BN, KP), lambda j: (j, 0)),
    out_shape=jax.ShapeDtypeStruct((NP, KP), jnp.float32),
    scratch_shapes=[pltpu.VMEM((KP, D), jnp.float32)],
)


def _sc_body(q_hbm, ei_hbm, zq_hbm, out_hbm, src_v, dst_v, rows_v, stage_v,
             acc_sh, sem):
    c = lax.axis_index("c")
    s = lax.axis_index("s")
    wid = s * NC + c

    row0 = pl.multiple_of(s * RPS, RPS)
    pltpu.sync_copy(zq_hbm.at[pl.ds(row0, RPS)], stage_v)
    pltpu.sync_copy(stage_v, acc_sh.at[pl.ds(row0, RPS)])
    plsc.subcore_barrier()

    base = pl.multiple_of(wid * EPW, CHUNK)

    def body(i, carry):
        eb = pl.multiple_of(base + i * CHUNK, CHUNK)
        pltpu.sync_copy(ei_hbm.at[pl.ds(eb, CHUNK)], src_v)
        pltpu.sync_copy(ei_hbm.at[pl.ds(E + eb, CHUNK)], dst_v)
        pltpu.async_copy(q_hbm.at[src_v], rows_v, sem).wait()
        pltpu.sync_copy(rows_v, acc_sh.at[dst_v], add=True)
        return carry

    lax.fori_loop(0, NCHUNK, body, 0, unroll=False)

    plsc.subcore_barrier()
    pltpu.sync_copy(acc_sh.at[pl.ds(row0, RPS)], stage_v)
    pltpu.sync_copy(stage_v, out_hbm.at[c, pl.ds(row0, RPS)])


@functools.cache
def _sc_segsum():
    return pl.kernel(
        _sc_body,
        out_type=jax.ShapeDtypeStruct((NC, NP, KP), jnp.float32),
        mesh=plsc.VectorSubcoreMesh(core_axis_name="c", subcore_axis_name="s",
                                    num_cores=NC, num_subcores=NS),
        scratch_types=[
            pltpu.VMEM((CHUNK,), jnp.int32),
            pltpu.VMEM((CHUNK,), jnp.int32),
            pltpu.VMEM((CHUNK, KP), jnp.float32),
            pltpu.VMEM((RPS, KP), jnp.float32),
            pltpu.VMEM_SHARED((NP, KP), jnp.float32),
            pltpu.SemaphoreType.DMA,
        ],
        compiler_params=pltpu.CompilerParams(use_tc_tiling_on_sc=False),
    )


def _stage_b_body(x_ref, m_ref, zp_ref, b_ref, out_ref,
                  num_ref, ws_ref, ctr_ref, cn2_ref):
    p = pl.program_id(0)
    j = pl.program_id(1)

    @pl.when((p == 0) & (j == 0))
    def _():
        num_ref[...] = jnp.zeros_like(num_ref)
        ws_ref[...] = jnp.zeros_like(ws_ref)

    @pl.when(p == 0)
    def _():
        z = zp_ref[0] + zp_ref[1]
        w = jax.nn.sigmoid(z)
        cw = m_ref[...] * w
        num_ref[...] += lax.dot_general(
            x_ref[...], cw, (((0,), (0,)), ((), ())),
            preferred_element_type=jnp.float32)
        ws_ref[...] += jnp.sum(w, axis=0, keepdims=True)

    @pl.when((p == 1) & (j == 0))
    def _():
        ctr = num_ref[...] / (ws_ref[...] + 1e-8)
        ctr_ref[...] = ctr
        cn2_ref[...] = jnp.sum(ctr * ctr, axis=0, keepdims=True)

    @pl.when(p == 1)
    def _():
        xb = x_ref[...]
        xc = jnp.dot(xb, ctr_ref[...], preferred_element_type=jnp.float32)
        sq = jnp.sum(xb * xb, axis=1, keepdims=True)
        score = 2.0 * xc - sq - cn2_ref[...] + b_ref[...]
        h = score > 0.0
        hf = h.astype(jnp.float32)
        ii = lax.broadcasted_iota(jnp.int32, (KP, KP), 0)
        jj = lax.broadcasted_iota(jnp.int32, (KP, KP), 1)
        tri = (ii <= jj).astype(jnp.float32)
        cnt = jnp.dot(hf, tri, preferred_element_type=jnp.float32)
        out_ref[...] = jnp.where(h & (cnt <= 2.0), 1.0, 0.0)


_stage_b = pl.pallas_call(
    _stage_b_body,
    grid=(2, NB),
    in_specs=[
        pl.BlockSpec((BN, D), lambda p, j: (j, 0)),
        pl.BlockSpec((BN, KP), lambda p, j: (j, 0)),
        pl.BlockSpec((NC, BN, KP), lambda p, j: (0, j, 0)),
        pl.BlockSpec((1, KP), lambda p, j: (0, 0)),
    ],
    out_specs=pl.BlockSpec((BN, KP), lambda p, j: (j, 0)),
    out_shape=jax.ShapeDtypeStruct((N, KP), jnp.float32),
    scratch_shapes=[
        pltpu.VMEM((D, KP), jnp.float32),
        pltpu.VMEM((1, KP), jnp.float32),
        pltpu.VMEM((D, KP), jnp.float32),
        pltpu.VMEM((1, KP), jnp.float32),
    ],
)


def kernel(x, edge_index, mask, W, v, b):
    mask_pad = jnp.pad(mask, ((0, 0), (0, KP - K)))
    b_pad = jnp.concatenate(
        [b, jnp.full((KP - K,), -jnp.inf, jnp.float32)]).reshape(1, KP)
    zq = jnp.zeros((NP, KP), jnp.float32)
    q = _stage_a(x, mask_pad, W, v)
    zparts = _sc_segsum()(q, edge_index.reshape(2 * E), zq)
    outp = _stage_b(x, mask_pad, zparts, b_pad)
    return outp[:, :K]


# speedup vs baseline: 48.0318x; 1.8446x over previous
import functools

import jax
import jax.numpy as jnp
from jax import lax
from jax.experimental import pallas as pl
from jax.experimental.pallas import tpu as pltpu
from jax.experimental.pallas import tpu_sc as plsc

N = 10000
D = 128
K = 8
KP = 16
E = 320000

NC = 2
NS = 16
NW = NC * NS
CHUNK = 128
EP = 327680
NCHT = EP // CHUNK
CPW = NCHT // NW
NP = 10240
RPS = NP // NS

BN = 1000
NB = N // BN


def _stage_a_body(x_ref, m_ref, w_ref, v_ref, q_ref, ut_ref):
    j = pl.program_id(0)

    @pl.when(j == 0)
    def _():
        u = lax.dot_general(w_ref[...], v_ref[...],
                            (((2,), (1,)), ((0,), (0,))),
                            preferred_element_type=jnp.float32)
        ut_ref[...] = jnp.concatenate(
            [u, jnp.zeros((KP - K, D), jnp.float32)], axis=0)

    p = lax.dot_general(x_ref[...], ut_ref[...], (((1,), (1,)), ((), ())),
                        preferred_element_type=jnp.float32)
    q_ref[...] = p * m_ref[...]


_stage_a = pl.pallas_call(
    _stage_a_body,
    grid=(NB,),
    in_specs=[
        pl.BlockSpec((BN, D), lambda j: (j, 0)),
        pl.BlockSpec((BN, KP), lambda j: (j, 0)),
        pl.BlockSpec((K, D, D), lambda j: (0, 0, 0)),
        pl.BlockSpec((K, D), lambda j: (0, 0)),
    ],
    out_specs=pl.BlockSpec((BN, KP), lambda j: (j, 0)),
    out_shape=jax.ShapeDtypeStruct((NP, KP), jnp.float32),
    scratch_shapes=[pltpu.VMEM((KP, D), jnp.float32)],
)


def _sc_body(q_hbm, ei3_hbm, zq_hbm, out_hbm, idxa, idxb, rowsa, rowsb,
             stage_v, acc_sh, semia, semib, semga, semgb):
    c = lax.axis_index("c")
    s = lax.axis_index("s")
    wid = s * NC + c

    row0 = pl.multiple_of(s * RPS, RPS)
    pltpu.sync_copy(zq_hbm.at[pl.ds(row0, RPS)], stage_v)
    pltpu.sync_copy(stage_v, acc_sh.at[pl.ds(row0, RPS)])
    plsc.subcore_barrier()

    c0 = pl.multiple_of(wid * CPW, CPW)

    def i_copy(ci, idx_v, sem):
        return pltpu.make_async_copy(ei3_hbm.at[ci], idx_v, sem)

    def g_copy(idx_v, rows_v, sem):
        return pltpu.make_async_copy(q_hbm.at[idx_v.at[0]], rows_v, sem)

    def s_add(idx_v, rows_v):
        pltpu.sync_copy(rows_v, acc_sh.at[idx_v.at[1]], add=True)

    i_copy(c0, idxa, semia).start()
    i_copy(c0 + 1, idxb, semib).start()
    i_copy(c0, idxa, semia).wait()
    g_copy(idxa, rowsa, semga).start()

    def body(t, carry):
        a = c0 + 2 * t
        i_copy(a + 1, idxb, semib).wait()
        g_copy(idxb, rowsb, semgb).start()
        g_copy(idxa, rowsa, semga).wait()
        s_add(idxa, rowsa)
        i_copy(a + 2, idxa, semia).start()
        g_copy(idxb, rowsb, semgb).wait()
        s_add(idxb, rowsb)
        i_copy(a + 3, idxb, semib).start()
        i_copy(a + 2, idxa, semia).wait()
        g_copy(idxa, rowsa, semga).start()
        return carry

    lax.fori_loop(0, CPW // 2 - 1, body, 0, unroll=False)

    i_copy(c0 + CPW - 1, idxb, semib).wait()
    g_copy(idxb, rowsb, semgb).start()
    g_copy(idxa, rowsa, semga).wait()
    s_add(idxa, rowsa)
    g_copy(idxb, rowsb, semgb).wait()
    s_add(idxb, rowsb)

    plsc.subcore_barrier()
    pltpu.sync_copy(acc_sh.at[pl.ds(row0, RPS)], stage_v)
    pltpu.sync_copy(stage_v, out_hbm.at[c, pl.ds(row0, RPS)])


@functools.cache
def _sc_segsum():
    return pl.kernel(
        _sc_body,
        out_type=jax.ShapeDtypeStruct((NC, NP, KP), jnp.float32),
        mesh=plsc.VectorSubcoreMesh(core_axis_name="c", subcore_axis_name="s",
                                    num_cores=NC, num_subcores=NS),
        scratch_types=[
            pltpu.VMEM((2, CHUNK), jnp.int32),
            pltpu.VMEM((2, CHUNK), jnp.int32),
            pltpu.VMEM((CHUNK, KP), jnp.float32),
            pltpu.VMEM((CHUNK, KP), jnp.float32),
            pltpu.VMEM((RPS, KP), jnp.float32),
            pltpu.VMEM_SHARED((NP, KP), jnp.float32),
            pltpu.SemaphoreType.DMA,
            pltpu.SemaphoreType.DMA,
            pltpu.SemaphoreType.DMA,
            pltpu.SemaphoreType.DMA,
        ],
        compiler_params=pltpu.CompilerParams(use_tc_tiling_on_sc=False),
    )


def _stage_b_body(x_ref, m_ref, zp_ref, b_ref, out_ref,
                  num_ref, ws_ref, ctr_ref, cn2_ref):
    p = pl.program_id(0)
    j = pl.program_id(1)

    @pl.when((p == 0) & (j == 0))
    def _():
        num_ref[...] = jnp.zeros_like(num_ref)
        ws_ref[...] = jnp.zeros_like(ws_ref)

    @pl.when(p == 0)
    def _():
        z = zp_ref[0] + zp_ref[1]
        w = jax.nn.sigmoid(z)
        cw = m_ref[...] * w
        num_ref[...] += lax.dot_general(
            x_ref[...], cw, (((0,), (0,)), ((), ())),
            preferred_element_type=jnp.float32)
        ws_ref[...] += jnp.sum(w, axis=0, keepdims=True)

    @pl.when((p == 1) & (j == 0))
    def _():
        ctr = num_ref[...] / (ws_ref[...] + 1e-8)
        ctr_ref[...] = ctr
        cn2_ref[...] = jnp.sum(ctr * ctr, axis=0, keepdims=True)

    @pl.when(p == 1)
    def _():
        xb = x_ref[...]
        xc = jnp.dot(xb, ctr_ref[...], preferred_element_type=jnp.float32)
        sq = jnp.sum(xb * xb, axis=1, keepdims=True)
        score = 2.0 * xc - sq - cn2_ref[...] + b_ref[...]
        h = score > 0.0
        hf = h.astype(jnp.float32)
        ii = lax.broadcasted_iota(jnp.int32, (KP, KP), 0)
        jj = lax.broadcasted_iota(jnp.int32, (KP, KP), 1)
        tri = (ii <= jj).astype(jnp.float32)
        cnt = jnp.dot(hf, tri, preferred_element_type=jnp.float32)
        out_ref[...] = jnp.where(h & (cnt <= 2.0), 1.0, 0.0)


_stage_b = pl.pallas_call(
    _stage_b_body,
    grid=(2, NB),
    in_specs=[
        pl.BlockSpec((BN, D), lambda p, j: (j, 0)),
        pl.BlockSpec((BN, KP), lambda p, j: (j, 0)),
        pl.BlockSpec((NC, BN, KP), lambda p, j: (0, j, 0)),
        pl.BlockSpec((1, KP), lambda p, j: (0, 0)),
    ],
    out_specs=pl.BlockSpec((BN, KP), lambda p, j: (j, 0)),
    out_shape=jax.ShapeDtypeStruct((N, KP), jnp.float32),
    scratch_shapes=[
        pltpu.VMEM((D, KP), jnp.float32),
        pltpu.VMEM((1, KP), jnp.float32),
        pltpu.VMEM((D, KP), jnp.float32),
        pltpu.VMEM((1, KP), jnp.float32),
    ],
)


def kernel(x, edge_index, mask, W, v, b):
    mask_pad = jnp.pad(mask, ((0, 0), (0, KP - K)))
    b_pad = jnp.concatenate(
        [b, jnp.full((KP - K,), -jnp.inf, jnp.float32)]).reshape(1, KP)
    zq = jnp.zeros((NP, KP), jnp.float32)
    pad = jnp.stack([jnp.zeros((EP - E,), jnp.int32),
                     jnp.full((EP - E,), N, jnp.int32)])
    ei3 = (jnp.concatenate([edge_index, pad], axis=1)
           .reshape(2, NCHT, CHUNK).transpose(1, 0, 2))
    q = _stage_a(x, mask_pad, W, v)
    zparts = _sc_segsum()(q, ei3, zq)
    outp = _stage_b(x, mask_pad, zparts, b_pad)
    return outp[:, :K]


# speedup vs baseline: 61.5117x; 1.2806x over previous
import functools

import jax
import jax.numpy as jnp
from jax import lax
from jax.experimental import pallas as pl
from jax.experimental.pallas import tpu as pltpu
from jax.experimental.pallas import tpu_sc as plsc

N = 10000
D = 128
K = 8
KP = 16
E = 320000

NC = 2
NS = 16
NW = NC * NS
CHUNK = 128
EP = 327680
NCHT = EP // CHUNK
CPW = NCHT // NW
NP = 10240
RPS = NP // NS

BN = 1000
NB = N // BN


def _stage_a_body(x_ref, m_ref, w_ref, v_ref, q_ref, ut_ref):
    j = pl.program_id(0)

    @pl.when(j == 0)
    def _():
        u = lax.dot_general(w_ref[...], v_ref[...],
                            (((2,), (1,)), ((0,), (0,))),
                            preferred_element_type=jnp.float32)
        ut_ref[...] = jnp.concatenate(
            [u, jnp.zeros((KP - K, D), jnp.float32)], axis=0)

    p = lax.dot_general(x_ref[...], ut_ref[...], (((1,), (1,)), ((), ())),
                        preferred_element_type=jnp.float32)
    q_ref[...] = p * m_ref[...]


_stage_a = pl.pallas_call(
    _stage_a_body,
    grid=(NB,),
    in_specs=[
        pl.BlockSpec((BN, D), lambda j: (j, 0)),
        pl.BlockSpec((BN, KP), lambda j: (j, 0)),
        pl.BlockSpec((K, D, D), lambda j: (0, 0, 0)),
        pl.BlockSpec((K, D), lambda j: (0, 0)),
    ],
    out_specs=pl.BlockSpec((BN, KP), lambda j: (j, 0)),
    out_shape=jax.ShapeDtypeStruct((NP, KP), jnp.float32),
    scratch_shapes=[pltpu.VMEM((KP, D), jnp.float32)],
)


def _sc_body(q_hbm, ei3_hbm, zq_hbm, out_hbm, idxa, idxb, rowsa, rowsb,
             stage_v, acc_sh, qtab_sh, semia, semib, semga, semgb):
    c = lax.axis_index("c")
    s = lax.axis_index("s")
    wid = s * NC + c

    row0 = pl.multiple_of(s * RPS, RPS)
    pltpu.sync_copy(q_hbm.at[pl.ds(row0, RPS)], stage_v)
    pltpu.sync_copy(stage_v, qtab_sh.at[pl.ds(row0, RPS)])
    pltpu.sync_copy(zq_hbm.at[pl.ds(row0, RPS)], stage_v)
    pltpu.sync_copy(stage_v, acc_sh.at[pl.ds(row0, RPS)])
    plsc.subcore_barrier()

    c0 = pl.multiple_of(wid * CPW, CPW)

    def i_copy(ci, idx_v, sem):
        return pltpu.make_async_copy(ei3_hbm.at[ci], idx_v, sem)

    def g_copy(idx_v, rows_v, sem):
        return pltpu.make_async_copy(qtab_sh.at[idx_v.at[0]], rows_v, sem)

    def s_add(idx_v, rows_v):
        pltpu.sync_copy(rows_v, acc_sh.at[idx_v.at[1]], add=True)

    i_copy(c0, idxa, semia).start()
    i_copy(c0 + 1, idxb, semib).start()
    i_copy(c0, idxa, semia).wait()
    g_copy(idxa, rowsa, semga).start()

    def body(t, carry):
        a = c0 + 2 * t
        i_copy(a + 1, idxb, semib).wait()
        g_copy(idxb, rowsb, semgb).start()
        g_copy(idxa, rowsa, semga).wait()
        s_add(idxa, rowsa)
        i_copy(a + 2, idxa, semia).start()
        g_copy(idxb, rowsb, semgb).wait()
        s_add(idxb, rowsb)
        i_copy(a + 3, idxb, semib).start()
        i_copy(a + 2, idxa, semia).wait()
        g_copy(idxa, rowsa, semga).start()
        return carry

    lax.fori_loop(0, CPW // 2 - 1, body, 0, unroll=False)

    i_copy(c0 + CPW - 1, idxb, semib).wait()
    g_copy(idxb, rowsb, semgb).start()
    g_copy(idxa, rowsa, semga).wait()
    s_add(idxa, rowsa)
    g_copy(idxb, rowsb, semgb).wait()
    s_add(idxb, rowsb)

    plsc.subcore_barrier()
    pltpu.sync_copy(acc_sh.at[pl.ds(row0, RPS)], stage_v)
    pltpu.sync_copy(stage_v, out_hbm.at[c, pl.ds(row0, RPS)])


@functools.cache
def _sc_segsum():
    return pl.kernel(
        _sc_body,
        out_type=jax.ShapeDtypeStruct((NC, NP, KP), jnp.float32),
        mesh=plsc.VectorSubcoreMesh(core_axis_name="c", subcore_axis_name="s",
                                    num_cores=NC, num_subcores=NS),
        scratch_types=[
            pltpu.VMEM((2, CHUNK), jnp.int32),
            pltpu.VMEM((2, CHUNK), jnp.int32),
            pltpu.VMEM((CHUNK, KP), jnp.float32),
            pltpu.VMEM((CHUNK, KP), jnp.float32),
            pltpu.VMEM((RPS, KP), jnp.float32),
            pltpu.VMEM_SHARED((NP, KP), jnp.float32),
            pltpu.VMEM_SHARED((NP, KP), jnp.float32),
            pltpu.SemaphoreType.DMA,
            pltpu.SemaphoreType.DMA,
            pltpu.SemaphoreType.DMA,
            pltpu.SemaphoreType.DMA,
        ],
        compiler_params=pltpu.CompilerParams(use_tc_tiling_on_sc=False),
    )


def _stage_b_body(x_ref, m_ref, zp_ref, b_ref, out_ref,
                  num_ref, ws_ref, ctr_ref, cn2_ref):
    p = pl.program_id(0)
    j = pl.program_id(1)

    @pl.when((p == 0) & (j == 0))
    def _():
        num_ref[...] = jnp.zeros_like(num_ref)
        ws_ref[...] = jnp.zeros_like(ws_ref)

    @pl.when(p == 0)
    def _():
        z = zp_ref[0] + zp_ref[1]
        w = jax.nn.sigmoid(z)
        cw = m_ref[...] * w
        num_ref[...] += lax.dot_general(
            x_ref[...], cw, (((0,), (0,)), ((), ())),
            preferred_element_type=jnp.float32)
        ws_ref[...] += jnp.sum(w, axis=0, keepdims=True)

    @pl.when((p == 1) & (j == 0))
    def _():
        ctr = num_ref[...] / (ws_ref[...] + 1e-8)
        ctr_ref[...] = ctr
        cn2_ref[...] = jnp.sum(ctr * ctr, axis=0, keepdims=True)

    @pl.when(p == 1)
    def _():
        xb = x_ref[...]
        xc = jnp.dot(xb, ctr_ref[...], preferred_element_type=jnp.float32)
        sq = jnp.sum(xb * xb, axis=1, keepdims=True)
        score = 2.0 * xc - sq - cn2_ref[...] + b_ref[...]
        h = score > 0.0
        hf = h.astype(jnp.float32)
        ii = lax.broadcasted_iota(jnp.int32, (KP, KP), 0)
        jj = lax.broadcasted_iota(jnp.int32, (KP, KP), 1)
        tri = (ii <= jj).astype(jnp.float32)
        cnt = jnp.dot(hf, tri, preferred_element_type=jnp.float32)
        out_ref[...] = jnp.where(h & (cnt <= 2.0), 1.0, 0.0)


_stage_b = pl.pallas_call(
    _stage_b_body,
    grid=(2, NB),
    in_specs=[
        pl.BlockSpec((BN, D), lambda p, j: (j, 0)),
        pl.BlockSpec((BN, KP), lambda p, j: (j, 0)),
        pl.BlockSpec((NC, BN, KP), lambda p, j: (0, j, 0)),
        pl.BlockSpec((1, KP), lambda p, j: (0, 0)),
    ],
    out_specs=pl.BlockSpec((BN, KP), lambda p, j: (j, 0)),
    out_shape=jax.ShapeDtypeStruct((N, KP), jnp.float32),
    scratch_shapes=[
        pltpu.VMEM((D, KP), jnp.float32),
        pltpu.VMEM((1, KP), jnp.float32),
        pltpu.VMEM((D, KP), jnp.float32),
        pltpu.VMEM((1, KP), jnp.float32),
    ],
)


def kernel(x, edge_index, mask, W, v, b):
    mask_pad = jnp.pad(mask, ((0, 0), (0, KP - K)))
    b_pad = jnp.concatenate(
        [b, jnp.full((KP - K,), -jnp.inf, jnp.float32)]).reshape(1, KP)
    zq = jnp.zeros((NP, KP), jnp.float32)
    pad = jnp.stack([jnp.zeros((EP - E,), jnp.int32),
                     jnp.full((EP - E,), N, jnp.int32)])
    ei3 = (jnp.concatenate([edge_index, pad], axis=1)
           .reshape(2, NCHT, CHUNK).transpose(1, 0, 2))
    q = _stage_a(x, mask_pad, W, v)
    zparts = _sc_segsum()(q, ei3, zq)
    outp = _stage_b(x, mask_pad, zparts, b_pad)
    return outp[:, :K]


# speedup vs baseline: 73.3658x; 1.1927x over previous
import functools

import jax
import jax.numpy as jnp
from jax import lax
from jax.experimental import pallas as pl
from jax.experimental.pallas import tpu as pltpu
from jax.experimental.pallas import tpu_sc as plsc

N = 10000
D = 128
K = 8
KP = 16
E = 320000

NC = 2
NS = 16
NW = NC * NS
EPW = E // NW
CHUNK = 128
MCH = 78
TAIL = EPW - MCH * CHUNK
NP = 10240
RPS = NP // NS


def _stage_a_body(x_ref, m_ref, w_ref, v_ref, q_ref):
    u = lax.dot_general(w_ref[...], v_ref[...], (((2,), (1,)), ((0,), (0,))),
                        preferred_element_type=jnp.float32)
    up = jnp.concatenate([u, jnp.zeros((KP - K, D), jnp.float32)], axis=0)
    q = lax.dot_general(x_ref[...], up, (((1,), (1,)), ((), ())),
                        preferred_element_type=jnp.float32)
    mp = jnp.concatenate(
        [m_ref[...], jnp.zeros((N, KP - K), jnp.float32)], axis=1)
    q_ref[...] = jnp.concatenate(
        [q * mp, jnp.zeros((NP - N, KP), jnp.float32)], axis=0)


_stage_a = pl.pallas_call(
    _stage_a_body,
    out_shape=jax.ShapeDtypeStruct((NP, KP), jnp.float32),
)


def _sc_body(q_hbm, ei_hbm, zq_hbm, out_hbm, idxa, idxb, idxt, rowsa, rowsb,
             rowst, stage_v, acc_sh, qtab_sh, semia, semib, semga, semgb):
    c = lax.axis_index("c")
    s = lax.axis_index("s")
    wid = s * NC + c

    row0 = pl.multiple_of(s * RPS, RPS)
    pltpu.sync_copy(q_hbm.at[pl.ds(row0, RPS)], stage_v)
    pltpu.sync_copy(stage_v, qtab_sh.at[pl.ds(row0, RPS)])
    pltpu.sync_copy(zq_hbm.at[pl.ds(row0, RPS)], stage_v)
    pltpu.sync_copy(stage_v, acc_sh.at[pl.ds(row0, RPS)])
    plsc.subcore_barrier()

    eb0 = pl.multiple_of(wid * EPW, 8)

    def i_copy(ci, idx_v, sem):
        off = pl.multiple_of(eb0 + ci * CHUNK, 8)
        return pltpu.make_async_copy(ei_hbm.at[:, pl.ds(off, CHUNK)],
                                     idx_v, sem)

    def g_copy(idx_v, rows_v, sem):
        return pltpu.make_async_copy(qtab_sh.at[idx_v.at[0]], rows_v, sem)

    def s_add(idx_v, rows_v):
        pltpu.sync_copy(rows_v, acc_sh.at[idx_v.at[1]], add=True)

    i_copy(0, idxa, semia).start()
    i_copy(1, idxb, semib).start()
    i_copy(0, idxa, semia).wait()
    g_copy(idxa, rowsa, semga).start()

    def body(t, carry):
        a = 2 * t
        i_copy(a + 1, idxb, semib).wait()
        g_copy(idxb, rowsb, semgb).start()
        g_copy(idxa, rowsa, semga).wait()
        s_add(idxa, rowsa)
        i_copy(a + 2, idxa, semia).start()
        g_copy(idxb, rowsb, semgb).wait()
        s_add(idxb, rowsb)
        i_copy(a + 3, idxb, semib).start()
        i_copy(a + 2, idxa, semia).wait()
        g_copy(idxa, rowsa, semga).start()
        return carry

    lax.fori_loop(0, MCH // 2 - 1, body, 0, unroll=False)

    i_copy(MCH - 1, idxb, semib).wait()
    g_copy(idxb, rowsb, semgb).start()
    g_copy(idxa, rowsa, semga).wait()
    s_add(idxa, rowsa)
    g_copy(idxb, rowsb, semgb).wait()
    s_add(idxb, rowsb)

    toff = pl.multiple_of(eb0 + MCH * CHUNK, 8)
    pltpu.sync_copy(ei_hbm.at[:, pl.ds(toff, TAIL)], idxt)
    pltpu.async_copy(qtab_sh.at[idxt.at[0]], rowst, semga).wait()
    pltpu.sync_copy(rowst, acc_sh.at[idxt.at[1]], add=True)

    plsc.subcore_barrier()
    pltpu.sync_copy(acc_sh.at[pl.ds(row0, RPS)], stage_v)
    pltpu.sync_copy(stage_v, out_hbm.at[c, pl.ds(row0, RPS)])


@functools.cache
def _sc_segsum():
    return pl.kernel(
        _sc_body,
        out_type=jax.ShapeDtypeStruct((NC, NP, KP), jnp.float32),
        mesh=plsc.VectorSubcoreMesh(core_axis_name="c", subcore_axis_name="s",
                                    num_cores=NC, num_subcores=NS),
        scratch_types=[
            pltpu.VMEM((2, CHUNK), jnp.int32),
            pltpu.VMEM((2, CHUNK), jnp.int32),
            pltpu.VMEM((2, TAIL), jnp.int32),
            pltpu.VMEM((CHUNK, KP), jnp.float32),
            pltpu.VMEM((CHUNK, KP), jnp.float32),
            pltpu.VMEM((TAIL, KP), jnp.float32),
            pltpu.VMEM((RPS, KP), jnp.float32),
            pltpu.VMEM_SHARED((NP, KP), jnp.float32),
            pltpu.VMEM_SHARED((NP, KP), jnp.float32),
            pltpu.SemaphoreType.DMA,
            pltpu.SemaphoreType.DMA,
            pltpu.SemaphoreType.DMA,
            pltpu.SemaphoreType.DMA,
        ],
        compiler_params=pltpu.CompilerParams(use_tc_tiling_on_sc=False),
    )


def _stage_b_body(x_ref, m_ref, zp_ref, b_ref, out_ref):
    z = zp_ref[0, :N] + zp_ref[1, :N]
    w = jax.nn.sigmoid(z)
    mp = jnp.concatenate(
        [m_ref[...], jnp.zeros((N, KP - K), jnp.float32)], axis=1)
    cw = mp * w
    xb = x_ref[...]
    num = lax.dot_general(xb, cw, (((0,), (0,)), ((), ())),
                          preferred_element_type=jnp.float32)
    ws = jnp.sum(w, axis=0, keepdims=True)
    ctr = num / (ws + 1e-8)
    cn2 = jnp.sum(ctr * ctr, axis=0, keepdims=True)
    xc = jnp.dot(xb, ctr, preferred_element_type=jnp.float32)
    sq = jnp.sum(xb * xb, axis=1, keepdims=True)
    bp = jnp.concatenate(
        [b_ref[...], jnp.full((1, KP - K), -jnp.inf, jnp.float32)], axis=1)
    score = 2.0 * xc - sq - cn2 + bp
    h = score > 0.0
    hf = h.astype(jnp.float32)
    ii = lax.broadcasted_iota(jnp.int32, (KP, KP), 0)
    jj = lax.broadcasted_iota(jnp.int32, (KP, KP), 1)
    tri = (ii <= jj).astype(jnp.float32)
    cnt = jnp.dot(hf, tri, preferred_element_type=jnp.float32)
    res = jnp.where(h & (cnt <= 2.0), 1.0, 0.0)
    out_ref[...] = res[:, :K]


_stage_b = pl.pallas_call(
    _stage_b_body,
    out_shape=jax.ShapeDtypeStruct((N, K), jnp.float32),
)


def kernel(x, edge_index, mask, W, v, b):
    zq = jnp.zeros((NP, KP), jnp.float32)
    q = _stage_a(x, mask, W, v)
    zparts = _sc_segsum()(q, edge_index, zq)
    return _stage_b(x, mask, zparts, b.reshape(1, K))


# speedup vs baseline: 87.7575x; 1.1962x over previous
import functools

import jax
import jax.numpy as jnp
from jax import lax
from jax.experimental import pallas as pl
from jax.experimental.pallas import tpu as pltpu
from jax.experimental.pallas import tpu_sc as plsc

N = 10000
D = 128
K = 8
KP = 16
E = 320000

NC = 2
NS = 16
NW = NC * NS
EPW = E // NW
CHUNK = 1000
MCH = EPW // CHUNK
NP = 10240
RPS = NP // NS


def _stage_a_body(x_ref, m_ref, w_ref, v_ref, q_ref):
    u = lax.dot_general(w_ref[...], v_ref[...], (((2,), (1,)), ((0,), (0,))),
                        preferred_element_type=jnp.float32)
    up = jnp.concatenate([u, jnp.zeros((KP - K, D), jnp.float32)], axis=0)
    q = lax.dot_general(x_ref[...], up, (((1,), (1,)), ((), ())),
                        preferred_element_type=jnp.float32)
    mp = jnp.concatenate(
        [m_ref[...], jnp.zeros((N, KP - K), jnp.float32)], axis=1)
    q_ref[...] = jnp.concatenate(
        [q * mp, jnp.zeros((NP - N, KP), jnp.float32)], axis=0)


_stage_a = pl.pallas_call(
    _stage_a_body,
    out_shape=jax.ShapeDtypeStruct((NP, KP), jnp.float32),
)


def _sc_body(q_hbm, ei_hbm, zq_hbm, out_hbm, idxa, idxb, rowsa, rowsb,
             stage_v, acc_sh, qtab_sh, semia, semib, semga, semgb):
    c = lax.axis_index("c")
    s = lax.axis_index("s")
    wid = s * NC + c

    row0 = pl.multiple_of(s * RPS, RPS)
    pltpu.sync_copy(q_hbm.at[pl.ds(row0, RPS)], stage_v)
    pltpu.sync_copy(stage_v, qtab_sh.at[pl.ds(row0, RPS)])
    pltpu.sync_copy(zq_hbm.at[pl.ds(row0, RPS)], stage_v)
    pltpu.sync_copy(stage_v, acc_sh.at[pl.ds(row0, RPS)])
    plsc.subcore_barrier()

    eb0 = pl.multiple_of(wid * EPW, 8)

    def i_copy(ci, idx_v, sem):
        off = pl.multiple_of(eb0 + ci * CHUNK, 8)
        return pltpu.make_async_copy(ei_hbm.at[:, pl.ds(off, CHUNK)],
                                     idx_v, sem)

    def g_copy(idx_v, rows_v, sem):
        return pltpu.make_async_copy(qtab_sh.at[idx_v.at[0]], rows_v, sem)

    def s_add(idx_v, rows_v):
        pltpu.sync_copy(rows_v, acc_sh.at[idx_v.at[1]], add=True)

    i_copy(0, idxa, semia).start()
    i_copy(1, idxb, semib).start()
    i_copy(0, idxa, semia).wait()
    g_copy(idxa, rowsa, semga).start()

    def body(t, carry):
        a = 2 * t
        i_copy(a + 1, idxb, semib).wait()
        g_copy(idxb, rowsb, semgb).start()
        g_copy(idxa, rowsa, semga).wait()
        s_add(idxa, rowsa)
        i_copy(a + 2, idxa, semia).start()
        g_copy(idxb, rowsb, semgb).wait()
        s_add(idxb, rowsb)
        i_copy(a + 3, idxb, semib).start()
        i_copy(a + 2, idxa, semia).wait()
        g_copy(idxa, rowsa, semga).start()
        return carry

    lax.fori_loop(0, MCH // 2 - 1, body, 0, unroll=False)

    i_copy(MCH - 1, idxb, semib).wait()
    g_copy(idxb, rowsb, semgb).start()
    g_copy(idxa, rowsa, semga).wait()
    s_add(idxa, rowsa)
    g_copy(idxb, rowsb, semgb).wait()
    s_add(idxb, rowsb)

    plsc.subcore_barrier()
    pltpu.sync_copy(acc_sh.at[pl.ds(row0, RPS)], stage_v)
    pltpu.sync_copy(stage_v, out_hbm.at[c, pl.ds(row0, RPS)])


@functools.cache
def _sc_segsum():
    return pl.kernel(
        _sc_body,
        out_type=jax.ShapeDtypeStruct((NC, NP, KP), jnp.float32),
        mesh=plsc.VectorSubcoreMesh(core_axis_name="c", subcore_axis_name="s",
                                    num_cores=NC, num_subcores=NS),
        scratch_types=[
            pltpu.VMEM((2, CHUNK), jnp.int32),
            pltpu.VMEM((2, CHUNK), jnp.int32),
            pltpu.VMEM((CHUNK, KP), jnp.float32),
            pltpu.VMEM((CHUNK, KP), jnp.float32),
            pltpu.VMEM((RPS, KP), jnp.float32),
            pltpu.VMEM_SHARED((NP, KP), jnp.float32),
            pltpu.VMEM_SHARED((NP, KP), jnp.float32),
            pltpu.SemaphoreType.DMA,
            pltpu.SemaphoreType.DMA,
            pltpu.SemaphoreType.DMA,
            pltpu.SemaphoreType.DMA,
        ],
        compiler_params=pltpu.CompilerParams(use_tc_tiling_on_sc=False),
    )


def _stage_b_body(x_ref, m_ref, zp_ref, b_ref, out_ref):
    z = zp_ref[0, :N] + zp_ref[1, :N]
    w = jax.nn.sigmoid(z)
    mp = jnp.concatenate(
        [m_ref[...], jnp.zeros((N, KP - K), jnp.float32)], axis=1)
    cw = mp * w
    xb = x_ref[...]
    num = lax.dot_general(xb, cw, (((0,), (0,)), ((), ())),
                          preferred_element_type=jnp.float32)
    ws = jnp.sum(w, axis=0, keepdims=True)
    ctr = num / (ws + 1e-8)
    cn2 = jnp.sum(ctr * ctr, axis=0, keepdims=True)
    xc = jnp.dot(xb, ctr, preferred_element_type=jnp.float32)
    sq = jnp.sum(xb * xb, axis=1, keepdims=True)
    bp = jnp.concatenate(
        [b_ref[...], jnp.full((1, KP - K), -jnp.inf, jnp.float32)], axis=1)
    score = 2.0 * xc - sq - cn2 + bp
    h = score > 0.0
    hf = h.astype(jnp.float32)
    ii = lax.broadcasted_iota(jnp.int32, (KP, KP), 0)
    jj = lax.broadcasted_iota(jnp.int32, (KP, KP), 1)
    tri = (ii <= jj).astype(jnp.float32)
    cnt = jnp.dot(hf, tri, preferred_element_type=jnp.float32)
    res = jnp.where(h & (cnt <= 2.0), 1.0, 0.0)
    out_ref[...] = res[:, :K]


_stage_b = pl.pallas_call(
    _stage_b_body,
    out_shape=jax.ShapeDtypeStruct((N, K), jnp.float32),
)


def kernel(x, edge_index, mask, W, v, b):
    zq = jnp.zeros((NP, KP), jnp.float32)
    q = _stage_a(x, mask, W, v)
    zparts = _sc_segsum()(q, edge_index, zq)
    return _stage_b(x, mask, zparts, b.reshape(1, K))


# speedup vs baseline: 100.0864x; 1.1405x over previous
import functools

import jax
import jax.numpy as jnp
from jax import lax
from jax.experimental import pallas as pl
from jax.experimental.pallas import tpu as pltpu
from jax.experimental.pallas import tpu_sc as plsc

N = 10000
D = 128
K = 8
KP = 16
E = 320000

NC = 2
NS = 16
NW = NC * NS
EPW = E // NW
CHUNK = 1000
MCH = EPW // CHUNK
NP = 10240
RPS = NP // NS


def _stage_a_body(x_ref, mt_ref, w_ref, v_ref, q_ref):
    u = lax.dot_general(w_ref[...], v_ref[...], (((2,), (1,)), ((0,), (0,))),
                        preferred_element_type=jnp.float32)
    up = jnp.concatenate([u, jnp.zeros((KP - K, D), jnp.float32)], axis=0)
    q = lax.dot_general(x_ref[...], up, (((1,), (1,)), ((), ())),
                        preferred_element_type=jnp.float32)
    m = lax.transpose(mt_ref[...], (1, 0))
    mp = jnp.concatenate([m, jnp.zeros((N, KP - K), jnp.float32)], axis=1)
    q_ref[...] = jnp.concatenate(
        [q * mp, jnp.zeros((NP - N, KP), jnp.float32)], axis=0)


_stage_a = pl.pallas_call(
    _stage_a_body,
    out_shape=jax.ShapeDtypeStruct((NP, KP), jnp.float32),
)


def _sc_body(q_hbm, ei_hbm, zq_hbm, out_hbm, idxa, idxb, rowsa, rowsb,
             stage_v, acc_sh, qtab_sh, semia, semib, semga, semgb):
    c = lax.axis_index("c")
    s = lax.axis_index("s")
    wid = s * NC + c

    row0 = pl.multiple_of(s * RPS, RPS)
    pltpu.sync_copy(q_hbm.at[pl.ds(row0, RPS)], stage_v)
    pltpu.sync_copy(stage_v, qtab_sh.at[pl.ds(row0, RPS)])
    pltpu.sync_copy(zq_hbm.at[pl.ds(row0, RPS)], stage_v)
    pltpu.sync_copy(stage_v, acc_sh.at[pl.ds(row0, RPS)])
    plsc.subcore_barrier()

    eb0 = pl.multiple_of(wid * EPW, 8)

    def i_copy(ci, idx_v, sem):
        off = pl.multiple_of(eb0 + ci * CHUNK, 8)
        return pltpu.make_async_copy(ei_hbm.at[:, pl.ds(off, CHUNK)],
                                     idx_v, sem)

    def g_copy(idx_v, rows_v, sem):
        return pltpu.make_async_copy(qtab_sh.at[idx_v.at[0]], rows_v, sem)

    def s_add(idx_v, rows_v):
        pltpu.sync_copy(rows_v, acc_sh.at[idx_v.at[1]], add=True)

    i_copy(0, idxa, semia).start()
    i_copy(1, idxb, semib).start()
    i_copy(0, idxa, semia).wait()
    g_copy(idxa, rowsa, semga).start()

    def body(t, carry):
        a = 2 * t
        i_copy(a + 1, idxb, semib).wait()
        g_copy(idxb, rowsb, semgb).start()
        g_copy(idxa, rowsa, semga).wait()
        s_add(idxa, rowsa)
        i_copy(a + 2, idxa, semia).start()
        g_copy(idxb, rowsb, semgb).wait()
        s_add(idxb, rowsb)
        i_copy(a + 3, idxb, semib).start()
        i_copy(a + 2, idxa, semia).wait()
        g_copy(idxa, rowsa, semga).start()
        return carry

    lax.fori_loop(0, MCH // 2 - 1, body, 0, unroll=False)

    i_copy(MCH - 1, idxb, semib).wait()
    g_copy(idxb, rowsb, semgb).start()
    g_copy(idxa, rowsa, semga).wait()
    s_add(idxa, rowsa)
    g_copy(idxb, rowsb, semgb).wait()
    s_add(idxb, rowsb)

    plsc.subcore_barrier()
    pltpu.sync_copy(acc_sh.at[pl.ds(row0, RPS)], stage_v)
    pltpu.sync_copy(stage_v, out_hbm.at[c, pl.ds(row0, RPS)])


@functools.cache
def _sc_segsum():
    return pl.kernel(
        _sc_body,
        out_type=jax.ShapeDtypeStruct((NC, NP, KP), jnp.float32),
        mesh=plsc.VectorSubcoreMesh(core_axis_name="c", subcore_axis_name="s",
                                    num_cores=NC, num_subcores=NS),
        scratch_types=[
            pltpu.VMEM((2, CHUNK), jnp.int32),
            pltpu.VMEM((2, CHUNK), jnp.int32),
            pltpu.VMEM((CHUNK, KP), jnp.float32),
            pltpu.VMEM((CHUNK, KP), jnp.float32),
            pltpu.VMEM((RPS, KP), jnp.float32),
            pltpu.VMEM_SHARED((NP, KP), jnp.float32),
            pltpu.VMEM_SHARED((NP, KP), jnp.float32),
            pltpu.SemaphoreType.DMA,
            pltpu.SemaphoreType.DMA,
            pltpu.SemaphoreType.DMA,
            pltpu.SemaphoreType.DMA,
        ],
        compiler_params=pltpu.CompilerParams(use_tc_tiling_on_sc=False),
    )


def _stage_b_body(x_ref, mt_ref, zp_ref, b_ref, out_ref):
    z = zp_ref[0, :N] + zp_ref[1, :N]
    w = jax.nn.sigmoid(z)
    m = lax.transpose(mt_ref[...], (1, 0))
    mp = jnp.concatenate([m, jnp.zeros((N, KP - K), jnp.float32)], axis=1)
    cw = mp * w
    xb = x_ref[...]
    num = lax.dot_general(xb, cw, (((0,), (0,)), ((), ())),
                          preferred_element_type=jnp.float32)
    ws = jnp.sum(w, axis=0, keepdims=True)
    ctr = num / (ws + 1e-8)
    cn2 = jnp.sum(ctr * ctr, axis=0, keepdims=True)
    xc = jnp.dot(xb, ctr, preferred_element_type=jnp.float32)
    sq = jnp.sum(xb * xb, axis=1, keepdims=True)
    bp = jnp.concatenate(
        [b_ref[...], jnp.full((1, KP - K), -jnp.inf, jnp.float32)], axis=1)
    score = 2.0 * xc - sq - cn2 + bp
    h = score > 0.0
    hf = h.astype(jnp.float32)
    ii = lax.broadcasted_iota(jnp.int32, (KP, KP), 0)
    jj = lax.broadcasted_iota(jnp.int32, (KP, KP), 1)
    tri = (ii <= jj).astype(jnp.float32)
    cnt = jnp.dot(hf, tri, preferred_element_type=jnp.float32)
    res = jnp.where(h & (cnt <= 2.0), 1.0, 0.0)
    out_ref[...] = lax.transpose(res[:, :K], (1, 0))


_stage_b = pl.pallas_call(
    _stage_b_body,
    out_shape=jax.ShapeDtypeStruct((K, N), jnp.float32),
)


def kernel(x, edge_index, mask, W, v, b):
    zq = jnp.zeros((NP, KP), jnp.float32)
    mt = mask.T
    q = _stage_a(x, mt, W, v)
    zparts = _sc_segsum()(q, edge_index, zq)
    outt = _stage_b(x, mt, zparts, b.reshape(1, K))
    return outt.T
